# Initial kernel scaffold; baseline (speedup 1.0000x reference)
import jax
import jax.numpy as jnp
from jax.experimental import pallas as pl


def kernel(x, W_att, a_att, W_out, a_out, edge_index):
    raise NotImplementedError("write your pallas kernel here")


# speedup vs baseline: 1.2353x; 1.2353x over previous
import functools

import jax
import jax.numpy as jnp
from jax.experimental import pallas as pl
from jax.experimental.pallas import tpu as pltpu

ALPHA = 0.1
NEG = -9e15


def _pick_block(n, candidates):
    for c in candidates:
        if n % c == 0:
            return c
    return n


def _leaky(v):
    return jnp.where(v > 0, v, ALPHA * v)


def _elu(v):
    return jnp.where(v > 0, v, jnp.exp(jnp.minimum(v, 0.0)) - 1.0)


def _prep1_kernel(nheads, nhid, x_ref, w_ref, a_ref, wh_ref, e1_ref, e2_ref):
    x = x_ref[...]
    for h in range(nheads):
        wh = jnp.dot(x, w_ref[h], preferred_element_type=jnp.float32)
        wh_ref[h] = wh
        e1_ref[:, h : h + 1] = jnp.dot(
            wh, a_ref[h, :nhid, :], preferred_element_type=jnp.float32
        )
        e2_ref[:, h : h + 1] = jnp.dot(
            wh, a_ref[h, nhid:, :], preferred_element_type=jnp.float32
        )


def _flash1_kernel(
    nheads,
    nhid,
    bc,
    br,
    n_real,
    npad,
    e1_ref,
    e2_ref,
    wh_ref,
    mask_ref,
    out_ref,
    acc_ref,
    m_ref,
    z_ref,
):
    i = pl.program_id(0)
    j = pl.program_id(1)
    nj = pl.num_programs(1)

    @pl.when(j == 0)
    def _():
        acc_ref[...] = jnp.zeros_like(acc_ref)
        m_ref[...] = jnp.full_like(m_ref, NEG)
        z_ref[...] = jnp.zeros_like(z_ref)

    mask = mask_ref[...] != 0
    e2t = e2_ref[...]
    for h in range(nheads):
        col = e1_ref[:, h : h + 1]
        row = e2t[:, h : h + 1].T
        v = _leaky(col + row)
        v = jnp.where(mask, v, NEG)
        m_old = m_ref[:, h : h + 1]
        m_new = jnp.maximum(m_old, jnp.max(v, axis=1, keepdims=True))
        corr = jnp.exp(m_old - m_new)
        p = jnp.exp(v - m_new)
        whb = wh_ref[h, pl.ds(j * bc, bc), :]
        acc_ref[:, h * nhid : (h + 1) * nhid] = acc_ref[
            :, h * nhid : (h + 1) * nhid
        ] * corr + jnp.dot(p, whb, preferred_element_type=jnp.float32)
        z_ref[:, h : h + 1] = z_ref[:, h : h + 1] * corr + jnp.sum(
            p, axis=1, keepdims=True
        )
        m_ref[:, h : h + 1] = m_new

    @pl.when(j == nj - 1)
    def _():
        rows = i * br + jax.lax.broadcasted_iota(jnp.int32, (br, 1), 0)
        rowok = rows < n_real
        for h in range(nheads):
            z = z_ref[:, h : h + 1] - npad * jnp.exp(NEG - m_ref[:, h : h + 1])
            t = acc_ref[:, h * nhid : (h + 1) * nhid] / z
            out_ref[:, h * nhid : (h + 1) * nhid] = jnp.where(rowok, _elu(t), 0.0)


def _prep2_kernel(nclass, h_ref, w_ref, a_ref, who_ref, e1_ref, e2_ref):
    who = jnp.dot(h_ref[...], w_ref[...], preferred_element_type=jnp.float32)
    who_ref[...] = who
    e1_ref[...] = jnp.dot(who, a_ref[:nclass, :], preferred_element_type=jnp.float32)
    e2_ref[...] = jnp.dot(who, a_ref[nclass:, :], preferred_element_type=jnp.float32)


def _flash2_kernel(
    bc, npad, e1_ref, e2_ref, who_ref, mask_ref, out_ref, acc_ref, m_ref, z_ref
):
    j = pl.program_id(1)
    nj = pl.num_programs(1)

    @pl.when(j == 0)
    def _():
        acc_ref[...] = jnp.zeros_like(acc_ref)
        m_ref[...] = jnp.full_like(m_ref, NEG)
        z_ref[...] = jnp.zeros_like(z_ref)

    mask = mask_ref[...] != 0
    v = _leaky(e1_ref[...] + e2_ref[...].T)
    v = jnp.where(mask, v, NEG)
    m_old = m_ref[...]
    m_new = jnp.maximum(m_old, jnp.max(v, axis=1, keepdims=True))
    corr = jnp.exp(m_old - m_new)
    p = jnp.exp(v - m_new)
    whb = who_ref[pl.ds(j * bc, bc), :]
    acc_ref[...] = acc_ref[...] * corr + jnp.dot(
        p, whb, preferred_element_type=jnp.float32
    )
    z_ref[...] = z_ref[...] * corr + jnp.sum(p, axis=1, keepdims=True)
    m_ref[...] = m_new

    @pl.when(j == nj - 1)
    def _():
        z = z_ref[...] - npad * jnp.exp(NEG - m_ref[...])
        t = _elu(acc_ref[...] / z)
        mx = jnp.max(t, axis=1, keepdims=True)
        lse = jnp.log(jnp.sum(jnp.exp(t - mx), axis=1, keepdims=True))
        out_ref[...] = t - mx - lse


def kernel(x, W_att, a_att, W_out, a_out, edge_index):
    n, nfeat = x.shape
    nheads, _, nhid = W_att.shape
    nclass = W_out.shape[1]

    bc = 2048 if n > 2048 else 128
    br = 512 if n > 512 else 8
    n2 = -(-n // bc) * bc
    npad = n2 - n
    ni, nj = n2 // br, n2 // bc

    if npad:
        x = jnp.zeros((n2, nfeat), x.dtype).at[:n].set(x)
    src, dst = edge_index[0], edge_index[1]
    mask = jnp.zeros((n2, n2), jnp.int8).at[src, dst].set(jnp.int8(1))

    wh, e1, e2 = pl.pallas_call(
        functools.partial(_prep1_kernel, nheads, nhid),
        grid=(ni,),
        in_specs=[
            pl.BlockSpec((br, nfeat), lambda i: (i, 0)),
            pl.BlockSpec((nheads, nfeat, nhid), lambda i: (0, 0, 0)),
            pl.BlockSpec((nheads, 2 * nhid, 1), lambda i: (0, 0, 0)),
        ],
        out_specs=[
            pl.BlockSpec((nheads, br, nhid), lambda i: (0, i, 0)),
            pl.BlockSpec((br, nheads), lambda i: (i, 0)),
            pl.BlockSpec((br, nheads), lambda i: (i, 0)),
        ],
        out_shape=[
            jax.ShapeDtypeStruct((nheads, n2, nhid), jnp.float32),
            jax.ShapeDtypeStruct((n2, nheads), jnp.float32),
            jax.ShapeDtypeStruct((n2, nheads), jnp.float32),
        ],
    )(x, W_att, a_att)

    h = pl.pallas_call(
        functools.partial(_flash1_kernel, nheads, nhid, bc, br, n, npad),
        grid=(ni, nj),
        in_specs=[
            pl.BlockSpec((br, nheads), lambda i, j: (i, 0)),
            pl.BlockSpec((bc, nheads), lambda i, j: (j, 0)),
            pl.BlockSpec((nheads, n2, nhid), lambda i, j: (0, 0, 0)),
            pl.BlockSpec((br, bc), lambda i, j: (i, j)),
        ],
        out_specs=pl.BlockSpec((br, nheads * nhid), lambda i, j: (i, 0)),
        out_shape=jax.ShapeDtypeStruct((n2, nheads * nhid), jnp.float32),
        scratch_shapes=[
            pltpu.VMEM((br, nheads * nhid), jnp.float32),
            pltpu.VMEM((br, nheads), jnp.float32),
            pltpu.VMEM((br, nheads), jnp.float32),
        ],
        compiler_params=pltpu.CompilerParams(
            dimension_semantics=("arbitrary", "arbitrary")
        ),
    )(e1, e2, wh, mask)

    who, e1o, e2o = pl.pallas_call(
        functools.partial(_prep2_kernel, nclass),
        grid=(ni,),
        in_specs=[
            pl.BlockSpec((br, nheads * nhid), lambda i: (i, 0)),
            pl.BlockSpec((nheads * nhid, nclass), lambda i: (0, 0)),
            pl.BlockSpec((2 * nclass, 1), lambda i: (0, 0)),
        ],
        out_specs=[
            pl.BlockSpec((br, nclass), lambda i: (i, 0)),
            pl.BlockSpec((br, 1), lambda i: (i, 0)),
            pl.BlockSpec((br, 1), lambda i: (i, 0)),
        ],
        out_shape=[
            jax.ShapeDtypeStruct((n2, nclass), jnp.float32),
            jax.ShapeDtypeStruct((n2, 1), jnp.float32),
            jax.ShapeDtypeStruct((n2, 1), jnp.float32),
        ],
    )(h, W_out, a_out)

    out = pl.pallas_call(
        functools.partial(_flash2_kernel, bc, npad),
        grid=(ni, nj),
        in_specs=[
            pl.BlockSpec((br, 1), lambda i, j: (i, 0)),
            pl.BlockSpec((bc, 1), lambda i, j: (j, 0)),
            pl.BlockSpec((n2, nclass), lambda i, j: (0, 0)),
            pl.BlockSpec((br, bc), lambda i, j: (i, j)),
        ],
        out_specs=pl.BlockSpec((br, nclass), lambda i, j: (i, 0)),
        out_shape=jax.ShapeDtypeStruct((n2, nclass), jnp.float32),
        scratch_shapes=[
            pltpu.VMEM((br, nclass), jnp.float32),
            pltpu.VMEM((br, 1), jnp.float32),
            pltpu.VMEM((br, 1), jnp.float32),
        ],
        compiler_params=pltpu.CompilerParams(
            dimension_semantics=("arbitrary", "arbitrary")
        ),
    )(e1o, e2o, who, mask)

    return out[:n] if npad else out


# speedup vs baseline: 2.0171x; 1.6329x over previous
import functools

import jax
import jax.numpy as jnp
from jax.experimental import pallas as pl
from jax.experimental.pallas import tpu as pltpu

ALPHA = 0.1
NEG = -9e15


def _pick_block(n, candidates):
    for c in candidates:
        if n % c == 0:
            return c
    return n


def _leaky(v):
    return jnp.maximum(v, ALPHA * v)


def _elu(v):
    return jnp.where(v > 0, v, jnp.exp(jnp.minimum(v, 0.0)) - 1.0)


def _prep1_kernel(nheads, nhid, x_ref, w_ref, a_ref, wh_ref, e1_ref, e2_ref):
    x = x_ref[...]
    for h in range(nheads):
        wh = jnp.dot(x, w_ref[h], preferred_element_type=jnp.float32)
        wh_ref[h] = wh
        e1_ref[:, h : h + 1] = jnp.dot(
            wh, a_ref[h, :nhid, :], preferred_element_type=jnp.float32
        )
        e2_ref[h : h + 1, :] = jnp.dot(
            wh, a_ref[h, nhid:, :], preferred_element_type=jnp.float32
        ).T


def _flash1_kernel(
    nheads,
    nhid,
    bc,
    br,
    n_real,
    npad,
    e1_ref,
    e2_ref,
    wh_ref,
    mask_ref,
    out_ref,
    acc_ref,
    m_ref,
    z_ref,
):
    i = pl.program_id(0)
    j = pl.program_id(1)
    nj = pl.num_programs(1)

    @pl.when(j == 0)
    def _():
        acc_ref[...] = jnp.zeros_like(acc_ref)
        m_ref[...] = jnp.full_like(m_ref, NEG)
        z_ref[...] = jnp.zeros_like(z_ref)

    mask = mask_ref[...] != 0
    for h in range(nheads):
        col = e1_ref[:, h : h + 1]
        row = e2_ref[h : h + 1, :]
        v = _leaky(col + row)
        v = jnp.where(mask, v, NEG)
        m_old = m_ref[:, h : h + 1]
        m_new = jnp.maximum(m_old, jnp.max(v, axis=1, keepdims=True))
        corr = jnp.exp(m_old - m_new)
        p = jnp.exp(v - m_new)
        whb = wh_ref[h, pl.ds(j * bc, bc), :]
        acc_ref[:, h * nhid : (h + 1) * nhid] = acc_ref[
            :, h * nhid : (h + 1) * nhid
        ] * corr + jnp.dot(p, whb, preferred_element_type=jnp.float32)
        z_ref[:, h : h + 1] = z_ref[:, h : h + 1] * corr + jnp.sum(
            p, axis=1, keepdims=True
        )
        m_ref[:, h : h + 1] = m_new

    @pl.when(j == nj - 1)
    def _():
        rows = i * br + jax.lax.broadcasted_iota(jnp.int32, (br, 1), 0)
        rowok = rows < n_real
        for h in range(nheads):
            z = z_ref[:, h : h + 1] - npad * jnp.exp(NEG - m_ref[:, h : h + 1])
            t = acc_ref[:, h * nhid : (h + 1) * nhid] / z
            out_ref[:, h * nhid : (h + 1) * nhid] = jnp.where(rowok, _elu(t), 0.0)


def _prep2_kernel(nclass, h_ref, w_ref, a_ref, who_ref, e1_ref, e2_ref):
    who = jnp.dot(h_ref[...], w_ref[...], preferred_element_type=jnp.float32)
    who_ref[...] = who
    e1_ref[...] = jnp.dot(who, a_ref[:nclass, :], preferred_element_type=jnp.float32)
    e2_ref[...] = jnp.dot(who, a_ref[nclass:, :], preferred_element_type=jnp.float32).T


def _flash2_kernel(
    bc, npad, e1_ref, e2_ref, who_ref, mask_ref, out_ref, acc_ref, m_ref, z_ref
):
    j = pl.program_id(1)
    nj = pl.num_programs(1)

    @pl.when(j == 0)
    def _():
        acc_ref[...] = jnp.zeros_like(acc_ref)
        m_ref[...] = jnp.full_like(m_ref, NEG)
        z_ref[...] = jnp.zeros_like(z_ref)

    mask = mask_ref[...] != 0
    v = _leaky(e1_ref[...] + e2_ref[...])
    v = jnp.where(mask, v, NEG)
    m_old = m_ref[...]
    m_new = jnp.maximum(m_old, jnp.max(v, axis=1, keepdims=True))
    corr = jnp.exp(m_old - m_new)
    p = jnp.exp(v - m_new)
    whb = who_ref[pl.ds(j * bc, bc), :]
    acc_ref[...] = acc_ref[...] * corr + jnp.dot(
        p, whb, preferred_element_type=jnp.float32
    )
    z_ref[...] = z_ref[...] * corr + jnp.sum(p, axis=1, keepdims=True)
    m_ref[...] = m_new

    @pl.when(j == nj - 1)
    def _():
        z = z_ref[...] - npad * jnp.exp(NEG - m_ref[...])
        t = _elu(acc_ref[...] / z)
        mx = jnp.max(t, axis=1, keepdims=True)
        lse = jnp.log(jnp.sum(jnp.exp(t - mx), axis=1, keepdims=True))
        out_ref[...] = t - mx - lse


def kernel(x, W_att, a_att, W_out, a_out, edge_index):
    n, nfeat = x.shape
    nheads, _, nhid = W_att.shape
    nclass = W_out.shape[1]

    bc = 2048 if n > 2048 else 128
    br = 512 if n > 512 else 8
    n2 = -(-n // bc) * bc
    npad = n2 - n
    ni, nj = n2 // br, n2 // bc

    if npad:
        x = jnp.zeros((n2, nfeat), x.dtype).at[:n].set(x)
    src, dst = edge_index[0], edge_index[1]
    flat = src * n2 + dst
    mask = (
        jnp.zeros((n2 * n2,), jnp.int32)
        .at[flat]
        .add(1, mode="promise_in_bounds")
        .reshape(n2, n2)
    )

    wh, e1, e2 = pl.pallas_call(
        functools.partial(_prep1_kernel, nheads, nhid),
        grid=(ni,),
        in_specs=[
            pl.BlockSpec((br, nfeat), lambda i: (i, 0)),
            pl.BlockSpec((nheads, nfeat, nhid), lambda i: (0, 0, 0)),
            pl.BlockSpec((nheads, 2 * nhid, 1), lambda i: (0, 0, 0)),
        ],
        out_specs=[
            pl.BlockSpec((nheads, br, nhid), lambda i: (0, i, 0)),
            pl.BlockSpec((br, nheads), lambda i: (i, 0)),
            pl.BlockSpec((nheads, br), lambda i: (0, i)),
        ],
        out_shape=[
            jax.ShapeDtypeStruct((nheads, n2, nhid), jnp.float32),
            jax.ShapeDtypeStruct((n2, nheads), jnp.float32),
            jax.ShapeDtypeStruct((nheads, n2), jnp.float32),
        ],
    )(x, W_att, a_att)

    h = pl.pallas_call(
        functools.partial(_flash1_kernel, nheads, nhid, bc, br, n, npad),
        grid=(ni, nj),
        in_specs=[
            pl.BlockSpec((br, nheads), lambda i, j: (i, 0)),
            pl.BlockSpec((nheads, bc), lambda i, j: (0, j)),
            pl.BlockSpec((nheads, n2, nhid), lambda i, j: (0, 0, 0)),
            pl.BlockSpec((br, bc), lambda i, j: (i, j)),
        ],
        out_specs=pl.BlockSpec((br, nheads * nhid), lambda i, j: (i, 0)),
        out_shape=jax.ShapeDtypeStruct((n2, nheads * nhid), jnp.float32),
        scratch_shapes=[
            pltpu.VMEM((br, nheads * nhid), jnp.float32),
            pltpu.VMEM((br, nheads), jnp.float32),
            pltpu.VMEM((br, nheads), jnp.float32),
        ],
        compiler_params=pltpu.CompilerParams(
            dimension_semantics=("arbitrary", "arbitrary")
        ),
    )(e1, e2, wh, mask)

    who, e1o, e2o = pl.pallas_call(
        functools.partial(_prep2_kernel, nclass),
        grid=(ni,),
        in_specs=[
            pl.BlockSpec((br, nheads * nhid), lambda i: (i, 0)),
            pl.BlockSpec((nheads * nhid, nclass), lambda i: (0, 0)),
            pl.BlockSpec((2 * nclass, 1), lambda i: (0, 0)),
        ],
        out_specs=[
            pl.BlockSpec((br, nclass), lambda i: (i, 0)),
            pl.BlockSpec((br, 1), lambda i: (i, 0)),
            pl.BlockSpec((1, br), lambda i: (0, i)),
        ],
        out_shape=[
            jax.ShapeDtypeStruct((n2, nclass), jnp.float32),
            jax.ShapeDtypeStruct((n2, 1), jnp.float32),
            jax.ShapeDtypeStruct((1, n2), jnp.float32),
        ],
    )(h, W_out, a_out)

    out = pl.pallas_call(
        functools.partial(_flash2_kernel, bc, npad),
        grid=(ni, nj),
        in_specs=[
            pl.BlockSpec((br, 1), lambda i, j: (i, 0)),
            pl.BlockSpec((1, bc), lambda i, j: (0, j)),
            pl.BlockSpec((n2, nclass), lambda i, j: (0, 0)),
            pl.BlockSpec((br, bc), lambda i, j: (i, j)),
        ],
        out_specs=pl.BlockSpec((br, nclass), lambda i, j: (i, 0)),
        out_shape=jax.ShapeDtypeStruct((n2, nclass), jnp.float32),
        scratch_shapes=[
            pltpu.VMEM((br, nclass), jnp.float32),
            pltpu.VMEM((br, 1), jnp.float32),
            pltpu.VMEM((br, 1), jnp.float32),
        ],
        compiler_params=pltpu.CompilerParams(
            dimension_semantics=("arbitrary", "arbitrary")
        ),
    )(e1o, e2o, who, mask)

    return out[:n] if npad else out


# speedup vs baseline: 2.1643x; 1.0730x over previous
import functools

import jax
import jax.numpy as jnp
from jax.experimental import pallas as pl
from jax.experimental.pallas import tpu as pltpu

ALPHA = 0.1
NEG = -9e15


def _pick_block(n, candidates):
    for c in candidates:
        if n % c == 0:
            return c
    return n


def _leaky(v):
    return jnp.maximum(v, ALPHA * v)


def _elu(v):
    return jnp.where(v > 0, v, jnp.exp(jnp.minimum(v, 0.0)) - 1.0)


def _prep1_kernel(nheads, nhid, x_ref, w_ref, a_ref, wh_ref, e1_ref, e2_ref):
    x = x_ref[...]
    for h in range(nheads):
        wh = jnp.dot(x, w_ref[h], preferred_element_type=jnp.float32)
        wh_ref[h] = wh
        e1_ref[:, h : h + 1] = jnp.dot(
            wh, a_ref[h, :nhid, :], preferred_element_type=jnp.float32
        )
        e2_ref[h : h + 1, :] = jnp.dot(
            wh, a_ref[h, nhid:, :], preferred_element_type=jnp.float32
        ).T


def _flash1_kernel(
    nheads,
    nhid,
    bc,
    br,
    n_real,
    e1_ref,
    e2_ref,
    wh_ref,
    mask_ref,
    out_ref,
    acc_ref,
    ws_ref,
    z_ref,
):
    i = pl.program_id(0)
    j = pl.program_id(1)
    nj = pl.num_programs(1)

    @pl.when(j == 0)
    def _():
        acc_ref[...] = jnp.zeros_like(acc_ref)
        z_ref[...] = jnp.zeros_like(z_ref)
        ws_ref[...] = jnp.zeros_like(ws_ref)

    mf = jnp.where(mask_ref[...] != 0, 1.0, 0.0)
    for h in range(nheads):
        col = e1_ref[:, h : h + 1]
        row = e2_ref[h : h + 1, :]
        p = jnp.exp(_leaky(col + row)) * mf
        whb = wh_ref[h, pl.ds(j * bc, bc), :]
        acc_ref[:, h * nhid : (h + 1) * nhid] += jnp.dot(
            p, whb, preferred_element_type=jnp.float32
        )
        z_ref[:, h : h + 1] += jnp.sum(p, axis=1, keepdims=True)
        ws_ref[h : h + 1, :] += jnp.sum(whb, axis=0, keepdims=True)

    @pl.when(j == nj - 1)
    def _():
        rows = i * br + jax.lax.broadcasted_iota(jnp.int32, (br, 1), 0)
        rowok = rows < n_real
        for h in range(nheads):
            z = z_ref[:, h : h + 1]
            t = jnp.where(
                z > 0,
                acc_ref[:, h * nhid : (h + 1) * nhid] / z,
                ws_ref[h : h + 1, :] / n_real,
            )
            out_ref[:, h * nhid : (h + 1) * nhid] = jnp.where(rowok, _elu(t), 0.0)


def _prep2_kernel(nclass, h_ref, w_ref, a_ref, who_ref, e1_ref, e2_ref):
    who = jnp.dot(h_ref[...], w_ref[...], preferred_element_type=jnp.float32)
    who_ref[...] = who
    e1_ref[...] = jnp.dot(who, a_ref[:nclass, :], preferred_element_type=jnp.float32)
    e2_ref[...] = jnp.dot(who, a_ref[nclass:, :], preferred_element_type=jnp.float32).T


def _flash2_kernel(
    bc, n_real, e1_ref, e2_ref, who_ref, mask_ref, out_ref, acc_ref, ws_ref, z_ref
):
    j = pl.program_id(1)
    nj = pl.num_programs(1)

    @pl.when(j == 0)
    def _():
        acc_ref[...] = jnp.zeros_like(acc_ref)
        z_ref[...] = jnp.zeros_like(z_ref)
        ws_ref[...] = jnp.zeros_like(ws_ref)

    mf = jnp.where(mask_ref[...] != 0, 1.0, 0.0)
    p = jnp.exp(_leaky(e1_ref[...] + e2_ref[...])) * mf
    whb = who_ref[pl.ds(j * bc, bc), :]
    acc_ref[...] += jnp.dot(p, whb, preferred_element_type=jnp.float32)
    z_ref[...] += jnp.sum(p, axis=1, keepdims=True)
    ws_ref[...] += jnp.sum(whb, axis=0, keepdims=True)

    @pl.when(j == nj - 1)
    def _():
        z = z_ref[...]
        t = _elu(jnp.where(z > 0, acc_ref[...] / z, ws_ref[...] / n_real))
        mx = jnp.max(t, axis=1, keepdims=True)
        lse = jnp.log(jnp.sum(jnp.exp(t - mx), axis=1, keepdims=True))
        out_ref[...] = t - mx - lse


def kernel(x, W_att, a_att, W_out, a_out, edge_index):
    n, nfeat = x.shape
    nheads, _, nhid = W_att.shape
    nclass = W_out.shape[1]

    bc = 2048 if n > 2048 else 128
    br = 512 if n > 512 else 8
    n2 = -(-n // bc) * bc
    npad = n2 - n
    ni, nj = n2 // br, n2 // bc

    if npad:
        x = jnp.zeros((n2, nfeat), x.dtype).at[:n].set(x)
    src, dst = edge_index[0], edge_index[1]
    flat = src * n2 + dst
    mask = (
        jnp.zeros((n2 * n2,), jnp.int32)
        .at[flat]
        .add(1, mode="promise_in_bounds")
        .reshape(n2, n2)
    )

    wh, e1, e2 = pl.pallas_call(
        functools.partial(_prep1_kernel, nheads, nhid),
        grid=(ni,),
        in_specs=[
            pl.BlockSpec((br, nfeat), lambda i: (i, 0)),
            pl.BlockSpec((nheads, nfeat, nhid), lambda i: (0, 0, 0)),
            pl.BlockSpec((nheads, 2 * nhid, 1), lambda i: (0, 0, 0)),
        ],
        out_specs=[
            pl.BlockSpec((nheads, br, nhid), lambda i: (0, i, 0)),
            pl.BlockSpec((br, nheads), lambda i: (i, 0)),
            pl.BlockSpec((nheads, br), lambda i: (0, i)),
        ],
        out_shape=[
            jax.ShapeDtypeStruct((nheads, n2, nhid), jnp.float32),
            jax.ShapeDtypeStruct((n2, nheads), jnp.float32),
            jax.ShapeDtypeStruct((nheads, n2), jnp.float32),
        ],
    )(x, W_att, a_att)

    h = pl.pallas_call(
        functools.partial(_flash1_kernel, nheads, nhid, bc, br, n),
        grid=(ni, nj),
        in_specs=[
            pl.BlockSpec((br, nheads), lambda i, j: (i, 0)),
            pl.BlockSpec((nheads, bc), lambda i, j: (0, j)),
            pl.BlockSpec((nheads, n2, nhid), lambda i, j: (0, 0, 0)),
            pl.BlockSpec((br, bc), lambda i, j: (i, j)),
        ],
        out_specs=pl.BlockSpec((br, nheads * nhid), lambda i, j: (i, 0)),
        out_shape=jax.ShapeDtypeStruct((n2, nheads * nhid), jnp.float32),
        scratch_shapes=[
            pltpu.VMEM((br, nheads * nhid), jnp.float32),
            pltpu.VMEM((nheads, nhid), jnp.float32),
            pltpu.VMEM((br, nheads), jnp.float32),
        ],
        compiler_params=pltpu.CompilerParams(
            dimension_semantics=("arbitrary", "arbitrary")
        ),
    )(e1, e2, wh, mask)

    who, e1o, e2o = pl.pallas_call(
        functools.partial(_prep2_kernel, nclass),
        grid=(ni,),
        in_specs=[
            pl.BlockSpec((br, nheads * nhid), lambda i: (i, 0)),
            pl.BlockSpec((nheads * nhid, nclass), lambda i: (0, 0)),
            pl.BlockSpec((2 * nclass, 1), lambda i: (0, 0)),
        ],
        out_specs=[
            pl.BlockSpec((br, nclass), lambda i: (i, 0)),
            pl.BlockSpec((br, 1), lambda i: (i, 0)),
            pl.BlockSpec((1, br), lambda i: (0, i)),
        ],
        out_shape=[
            jax.ShapeDtypeStruct((n2, nclass), jnp.float32),
            jax.ShapeDtypeStruct((n2, 1), jnp.float32),
            jax.ShapeDtypeStruct((1, n2), jnp.float32),
        ],
    )(h, W_out, a_out)

    out = pl.pallas_call(
        functools.partial(_flash2_kernel, bc, n),
        grid=(ni, nj),
        in_specs=[
            pl.BlockSpec((br, 1), lambda i, j: (i, 0)),
            pl.BlockSpec((1, bc), lambda i, j: (0, j)),
            pl.BlockSpec((n2, nclass), lambda i, j: (0, 0)),
            pl.BlockSpec((br, bc), lambda i, j: (i, j)),
        ],
        out_specs=pl.BlockSpec((br, nclass), lambda i, j: (i, 0)),
        out_shape=jax.ShapeDtypeStruct((n2, nclass), jnp.float32),
        scratch_shapes=[
            pltpu.VMEM((br, nclass), jnp.float32),
            pltpu.VMEM((1, nclass), jnp.float32),
            pltpu.VMEM((br, 1), jnp.float32),
        ],
        compiler_params=pltpu.CompilerParams(
            dimension_semantics=("arbitrary", "arbitrary")
        ),
    )(e1o, e2o, who, mask)

    return out[:n] if npad else out
